# Initial kernel scaffold; baseline (speedup 1.0000x reference)
import jax
import jax.numpy as jnp
from jax.experimental import pallas as pl


def kernel(x, enc_mask_token):
    raise NotImplementedError("write your pallas kernel here")


# speedup vs baseline: 21.7572x; 21.7572x over previous
import functools

import jax
import jax.numpy as jnp
import numpy as np
from jax import lax
from jax.experimental import pallas as pl
from jax.experimental.pallas import tpu as pltpu
from jax.experimental.pallas import tpu_sc as plsc

N = 100000
IN_DIM = 128
MASK_RATE = 0.5
REPLACE_RATE = 0.05

NC = 2
NS = 16
NW = NC * NS
CHUNK = 128


def _threefry2x32_np(k1, k2, x1, x2):
    u32 = np.uint32

    def rotl(x, d):
        return (x << u32(d)) | (x >> u32(32 - d))

    def rounds(x0, x1, rots):
        for r in rots:
            x0 = x0 + x1
            x1 = x0 ^ rotl(x1, r)
        return x0, x1

    ra, rb = (13, 15, 26, 6), (17, 29, 16, 24)
    ks = (u32(k1), u32(k2), u32(k1) ^ u32(k2) ^ u32(0x1BD11BDA))
    x0 = x1_ = None
    x0, x1_ = u32(x1) + ks[0], u32(x2) + ks[1]
    x0, x1_ = rounds(x0, x1_, ra)
    x0, x1_ = x0 + ks[1], x1_ + ks[2] + u32(1)
    x0, x1_ = rounds(x0, x1_, rb)
    x0, x1_ = x0 + ks[2], x1_ + ks[0] + u32(2)
    x0, x1_ = rounds(x0, x1_, ra)
    x0, x1_ = x0 + ks[0], x1_ + ks[1] + u32(3)
    x0, x1_ = rounds(x0, x1_, rb)
    x0, x1_ = x0 + ks[1], x1_ + ks[2] + u32(4)
    x0, x1_ = rounds(x0, x1_, ra)
    x0, x1_ = x0 + ks[2], x1_ + ks[0] + u32(5)
    return x0, x1_


def _np_split(key, num):
    hi = np.zeros(num, np.uint32)
    lo = np.arange(num, dtype=np.uint32)
    b1, b2 = _threefry2x32_np(key[0], key[1], hi, lo)
    return [(b1[i], b2[i]) for i in range(num)]


def _np_random_bits32(key, n):
    hi = np.zeros(n, np.uint32)
    lo = np.arange(n, dtype=np.uint32)
    b1, b2 = _threefry2x32_np(key[0], key[1], hi, lo)
    return b1 ^ b2


def _np_permutation(key, n):
    exponent = 3
    num_rounds = int(np.ceil(exponent * np.log(max(1, n))
                             / np.log(np.iinfo(np.uint32).max)))
    x = np.arange(n, dtype=np.int32)
    for _ in range(num_rounds):
        key, subkey = _np_split(key, 2)
        sort_keys = _np_random_bits32(subkey, n)
        x = x[np.argsort(sort_keys, kind="stable")]
    return x


def _build_index_constants():
    rk = (np.uint32(0), np.uint32(42))
    k1, k2, k3 = _np_split(rk, 3)
    perm = _np_permutation(k1, N)
    num_mask = int(MASK_RATE * N)
    num_noise = int(REPLACE_RATE * num_mask)
    perm_mask = _np_permutation(k2, num_mask)
    noise_src = _np_permutation(k3, N)[:num_noise]
    mask_nodes = perm[:num_mask]
    keep_nodes = perm[num_mask:]
    token_nodes = mask_nodes[perm_mask[:-num_noise]]
    noise_nodes = mask_nodes[perm_mask[-num_noise:]]

    cdst = np.concatenate([keep_nodes, noise_nodes])
    csrc = np.concatenate([keep_nodes, noise_src])
    order = np.argsort(cdst, kind="stable")
    cdst, csrc = cdst[order], csrc[order]

    tdst = np.sort(token_nodes)

    def partition_pad(dst, src, workers, chunk):
        n = len(dst)
        base, rem = divmod(n, workers)
        per_w = -(-(base + 1) // chunk) * chunk
        out_d = np.empty((workers, per_w // chunk, chunk), np.int32)
        out_s = np.empty_like(out_d)
        pos = 0
        for w in range(workers):
            take = base + (1 if w < rem else 0)
            d = dst[pos:pos + take]
            s = src[pos:pos + take]
            pos += take
            pad = per_w - take
            d = np.concatenate([d, np.full(pad, d[-1], d.dtype)])
            s = np.concatenate([s, np.full(pad, s[-1], s.dtype)])
            out_d[w] = d.reshape(-1, chunk)
            out_s[w] = s.reshape(-1, chunk)
        return out_d, out_s

    cdst_w, csrc_w = partition_pad(cdst, csrc, NW, CHUNK)
    tdst_w, _ = partition_pad(tdst, tdst, NW, CHUNK)

    return (mask_nodes.astype(np.int32), keep_nodes.astype(np.int32),
            cdst_w, csrc_w, tdst_w)


(_MASK_NODES, _KEEP_NODES, _CDST, _CSRC, _TDST) = _build_index_constants()
_C_CHUNKS = _CDST.shape[1]
_T_CHUNKS = _TDST.shape[1]


def _sc_body(x_hbm, tok_hbm, csrc_hbm, cdst_hbm, tdst_hbm, out_hbm,
             csrc_v, cdst_v, tdst_v, rows0_v, rows1_v, tok_v,
             gsem, ssem, tsem):
    w = lax.axis_index("s") * NC + lax.axis_index("c")

    pltpu.sync_copy(csrc_hbm.at[w], csrc_v)
    pltpu.sync_copy(cdst_hbm.at[w], cdst_v)
    pltpu.sync_copy(tdst_hbm.at[w], tdst_v)
    pltpu.sync_copy(tok_hbm, tok_v)

    tok_copies = [
        pltpu.async_copy(tok_v, out_hbm.at[tdst_v.at[j]], tsem)
        for j in range(_T_CHUNKS)
    ]

    bufs = (rows0_v, rows1_v)
    gather = [None] * _C_CHUNKS
    scatter = [None] * _C_CHUNKS
    gather[0] = pltpu.async_copy(x_hbm.at[csrc_v.at[0]], bufs[0], gsem)
    for j in range(_C_CHUNKS):
        buf = bufs[j % 2]
        gather[j].wait()
        if j + 1 < _C_CHUNKS:
            if j >= 1:
                scatter[j - 1].wait()
            gather[j + 1] = pltpu.async_copy(
                x_hbm.at[csrc_v.at[j + 1]], bufs[(j + 1) % 2], gsem)
        scatter[j] = pltpu.async_copy(buf, out_hbm.at[cdst_v.at[j]], ssem)
    scatter[_C_CHUNKS - 1].wait()
    if _C_CHUNKS >= 2:
        scatter[_C_CHUNKS - 2].wait()
    for c in tok_copies:
        c.wait()


@functools.partial(jax.jit, static_argnames=())
def _sc_call(x, tok_block, csrc, cdst, tdst):
    mesh = plsc.VectorSubcoreMesh(
        core_axis_name="c", subcore_axis_name="s",
        num_cores=NC, num_subcores=NS)
    f = pl.kernel(
        _sc_body,
        out_type=jax.ShapeDtypeStruct((N, IN_DIM), jnp.float32),
        mesh=mesh,
        scratch_types=[
            pltpu.VMEM((_C_CHUNKS, CHUNK), jnp.int32),
            pltpu.VMEM((_C_CHUNKS, CHUNK), jnp.int32),
            pltpu.VMEM((_T_CHUNKS, CHUNK), jnp.int32),
            pltpu.VMEM((CHUNK, IN_DIM), jnp.float32),
            pltpu.VMEM((CHUNK, IN_DIM), jnp.float32),
            pltpu.VMEM((CHUNK, IN_DIM), jnp.float32),
            pltpu.SemaphoreType.DMA,
            pltpu.SemaphoreType.DMA,
            pltpu.SemaphoreType.DMA,
        ],
    )
    return f(x, tok_block, csrc, cdst, tdst)


def kernel(x, enc_mask_token):
    tok_block = jnp.tile(enc_mask_token.astype(jnp.float32), (CHUNK, 1))
    out = _sc_call(
        x,
        tok_block,
        jnp.asarray(_CSRC),
        jnp.asarray(_CDST),
        jnp.asarray(_TDST),
    )
    return (out, jnp.asarray(_MASK_NODES), jnp.asarray(_KEEP_NODES))
